# baseline (speedup 1.0000x reference)
import functools

import jax
import jax.numpy as jnp
import numpy as np
from jax.experimental import pallas as pl
from jax.experimental.pallas import tpu as pltpu
from jax.experimental.pallas import tpu_sc as plsc

N0 = 5064
NR, NL = 40, 128
NP = NR * NL
NGT = 64
NSAMP = 128
NWORK = 8


def _np_threefry(k0, k1, x0, x1):
    u32 = np.uint32
    rotations = ((13, 15, 26, 6), (17, 29, 16, 24))
    with np.errstate(over="ignore"):
        ks = (k0, k1, u32(k0 ^ k1 ^ u32(0x1BD11BDA)))
        x0 = (x0 + ks[0]).astype(u32)
        x1 = (x1 + ks[1]).astype(u32)
        for i in range(5):
            for r in rotations[i % 2]:
                x0 = (x0 + x1).astype(u32)
                x1 = ((x1 << u32(r)) | (x1 >> u32(32 - r))).astype(u32)
                x1 = (x0 ^ x1).astype(u32)
            x0 = (x0 + ks[(i + 1) % 3]).astype(u32)
            x1 = (x1 + ks[(i + 2) % 3] + u32(i + 1)).astype(u32)
    return x0, x1


def _np_split(kd):
    c1 = np.zeros(2, np.uint32)
    c2 = np.arange(2, dtype=np.uint32)
    b1, b2 = _np_threefry(kd[0], kd[1], c1, c2)
    return np.array([b1[0], b2[0]], np.uint32), np.array([b1[1], b2[1]], np.uint32)


def _np_bits(kd, n):
    j = np.arange(n, dtype=np.uint32)
    o0, o1 = _np_threefry(kd[0], kd[1], np.zeros(n, np.uint32), j)
    return o0 ^ o1


def _const_streams():
    out = []
    for seed in (1, 2):
        key = np.array([0, seed], np.uint32)
        key1, s1 = _np_split(key)
        _, s2 = _np_split(key1)
        for s in (s1, s2):
            b = _np_bits(s, N0)
            o = np.argsort(b, kind="stable").astype(np.int32)
            o = np.concatenate([o, np.full((NP - N0,), 100000, np.int32)])
            out.append(o.reshape(NR, NL))
    return out


_ORD1P, _ORD2P, _ORD1N, _ORD2N = _const_streams()


def _cumsum2d(x):
    for s in (1, 2, 4, 8, 16, 32, 64):
        x = x + jnp.concatenate(
            [jnp.zeros((NR, s), x.dtype), x[:, :NL - s]], axis=1)
    rt = x[:, NL - 1:NL]
    rc = rt
    for s in (1, 2, 4, 8, 16, 32):
        rc = rc + jnp.concatenate(
            [jnp.zeros((s, 1), x.dtype), rc[:NR - s, :]], axis=0)
    return x + (rc - rt)


def _bitsel(ordv, m, c, targets):
    cm = jnp.where(m, c, 0)
    hit = cm[None, :, :] == targets[:, None, None] + 1
    return jnp.sum(jnp.where(hit, ordv[None, :, :], 0), axis=(1, 2))


def _branch(mask, vm, idx2d, n, ord1, ord2, ktake):
    nm = (~mask) & vm
    cP = _cumsum2d(mask.astype(jnp.int32))
    cN = _cumsum2d(nm.astype(jnp.int32))
    p = jnp.where(mask, cP - 1, n + cN - 1)
    p = jnp.where(vm, p, 6000 + idx2d)
    rvec = jax.lax.iota(jnp.int32, ktake)
    m1 = ord1 < n
    c1 = _cumsum2d(m1.astype(jnp.int32))

    def one_round():
        return jnp.where(rvec < n, _bitsel(ord1, m1, c1, rvec), rvec)

    def two_rounds():
        m2 = ord2 < n
        c2 = _cumsum2d(m2.astype(jnp.int32))
        qt = jnp.where(rvec < n, _bitsel(ord2, m2, c2, rvec), rvec)
        return jnp.where(qt < n, _bitsel(ord1, m1, c1, qt), qt)

    pstar = jax.lax.cond(n > 1625, two_rounds, one_round)
    hit = p[None, :, :] == pstar[:, None, None]
    return jnp.sum(jnp.where(hit, idx2d[None, :, :], 0), axis=(1, 2))


def _tc_kernel(rois_ref, gt_ref, lbl_ref, o1p_ref, o2p_ref, o1n_ref, o2n_ref,
               keep_ref, tab_ref):
    x1 = rois_ref[0]
    y1 = rois_ref[1]
    x2 = rois_ref[2]
    y2 = rois_ref[3]
    gx1 = gt_ref[:, 0].reshape(NGT, 1, 1)
    gy1 = gt_ref[:, 1].reshape(NGT, 1, 1)
    gx2 = gt_ref[:, 2].reshape(NGT, 1, 1)
    gy2 = gt_ref[:, 3].reshape(NGT, 1, 1)

    iw = jnp.clip(jnp.minimum(gx2, x2[None]) - jnp.maximum(gx1, x1[None]), 0.0, None)
    ih = jnp.clip(jnp.minimum(gy2, y2[None]) - jnp.maximum(gy1, y1[None]), 0.0, None)
    inter = iw * ih
    area_r = (x2 - x1) * (y2 - y1)
    area_g = (gx2 - gx1) * (gy2 - gy1)
    union = area_r[None] + area_g - inter
    iou = inter / union

    maxv = jnp.max(iou, axis=0)
    g_iota = jax.lax.broadcasted_iota(jnp.int32, (NGT, NR, NL), 0)
    am = jnp.min(jnp.where(iou == maxv[None], g_iota, NGT), axis=0)

    idx2d = jax.lax.broadcasted_iota(jnp.int32, (NR, NL), 0) * NL + \
        jax.lax.broadcasted_iota(jnp.int32, (NR, NL), 1)
    vm = idx2d < N0
    pos_mask = (maxv >= 0.5) & vm
    neg_mask = (maxv < 0.5) & (maxv >= 0.0) & vm

    n_pos = jnp.sum(pos_mask.astype(jnp.int32))
    n_neg = jnp.sum(neg_mask.astype(jnp.int32))
    n_pos_t = jnp.minimum(jnp.sum(((maxv > 0.5) & vm).astype(jnp.int32)), 32)

    pos_roi = _branch(pos_mask, vm, idx2d, n_pos, o1p_ref[...], o2p_ref[...], 32)
    neg_roi = _branch(neg_mask, vm, idx2d, n_neg, o1n_ref[...], o2n_ref[...], 96)
    keep = jnp.concatenate([pos_roi, neg_roi])

    keep_ref[...] = jnp.concatenate(
        [keep.reshape(1, NSAMP),
         jnp.full((1, NSAMP), n_pos_t, jnp.int32),
         jnp.zeros((6, NSAMP), jnp.int32)], axis=0)

    hit2 = g_iota == am[None]

    def gsel(gv):
        return jnp.sum(jnp.where(hit2, gv, 0.0), axis=0)

    gx1d, gy1d, gx2d, gy2d = gsel(gx1), gsel(gy1), gsel(gx2), gsel(gy2)
    lblf = lbl_ref[0, :].astype(jnp.float32).reshape(NGT, 1, 1)
    clsf = gsel(lblf) + 1.0

    pw = x2 - x1
    ph = y2 - y1
    pcx = (x1 + x2) / 2.0
    pcy = (y1 + y2) / 2.0
    gw = gx2d - gx1d
    gh = gy2d - gy1d
    gcx = (gx1d + gx2d) / 2.0
    gcy = (gy1d + gy2d) / 2.0
    tx = (gcx - pcx) / pw
    ty = (gcy - pcy) / ph
    tw = jnp.log(gw / pw)
    th = jnp.log(gh / ph)

    zero = jnp.zeros((NR, NL), jnp.float32)
    tab_ref[...] = jnp.stack(
        [tx, ty, tw, th, x1, y1, x2, y2, clsf,
         zero, zero, zero, zero, zero, zero, zero], axis=0)


def _sc_kernel(keep_h, tab_h, cls_h, loc_h, sroi_h,
               keep_v, npos_v, idx_v, rows_v, cls_v, loc_v, sroi_v, sem):
    w = jax.lax.axis_index("s") * 2 + jax.lax.axis_index("c")

    @pl.when(w < NWORK)
    def _():
        pltpu.sync_copy(keep_h.at[0, pl.ds(w * 16, 16)], keep_v)
        pltpu.sync_copy(keep_h.at[1, pl.ds(0, 16)], npos_v)
        keep = keep_v[...]
        idx_v[...] = jnp.right_shift(keep, 3)
        pltpu.async_copy(tab_h.at[idx_v], rows_v, sem).wait()

        lane = jax.lax.iota(jnp.int32, 16)
        coloff = jnp.bitwise_and(keep, 7) * 16

        def col(j):
            return plsc.load_gather(rows_v, [lane, coloff + j])

        tx, ty, tw, th = col(0), col(1), col(2), col(3)
        x1, y1, x2, y2 = col(4), col(5), col(6), col(7)
        clsi = col(8).astype(jnp.int32)

        sidx = w * 16 + lane
        cls = jnp.where(sidx < npos_v[...], clsi, 0)

        cls_v[...] = cls
        for j, v in enumerate((tx, ty, tw, th)):
            plsc.store_scatter(loc_v, [lane, jnp.full((16,), j, jnp.int32)], v)
        for j, v in enumerate((x1, y1, x2, y2)):
            plsc.store_scatter(sroi_v, [lane, jnp.full((16,), j, jnp.int32)], v)

        pltpu.sync_copy(cls_v, cls_h.at[pl.ds(w * 16, 16)])
        pltpu.sync_copy(loc_v, loc_h.at[pl.ds(w * 16, 16), :])
        pltpu.sync_copy(sroi_v, sroi_h.at[pl.ds(w * 16, 16), :])


def _sc_assemble(keepinfo, tab16, interpret=False):
    mesh = plsc.VectorSubcoreMesh(
        core_axis_name="c", subcore_axis_name="s", num_cores=2, num_subcores=16)
    f = pl.kernel(
        _sc_kernel,
        out_type=[
            jax.ShapeDtypeStruct((NSAMP,), jnp.int32),
            jax.ShapeDtypeStruct((NSAMP, 4), jnp.float32),
            jax.ShapeDtypeStruct((NSAMP, 4), jnp.float32),
        ],
        mesh=mesh,
        compiler_params=pltpu.CompilerParams(needs_layout_passes=False),
        scratch_types=[
            pltpu.VMEM((16,), jnp.int32),
            pltpu.VMEM((16,), jnp.int32),
            pltpu.VMEM((16,), jnp.int32),
            pltpu.VMEM((16, 128), jnp.float32),
            pltpu.VMEM((16,), jnp.int32),
            pltpu.VMEM((16, 4), jnp.float32),
            pltpu.VMEM((16, 4), jnp.float32),
            pltpu.SemaphoreType.DMA,
        ],
        interpret=interpret,
    )
    return f(keepinfo, tab16)


@functools.partial(jax.jit, static_argnames=("interpret",))
def _run(bbox, label, rois, interpret=False):
    bbox = bbox[0]
    label = label[0]
    rois_all = jnp.concatenate([rois, bbox], axis=0)
    rois_pad = jnp.concatenate(
        [rois_all, jnp.zeros((NP - N0, 4), jnp.float32)], axis=0)
    rois_pl = rois_pad.T.reshape(4, NR, NL)
    gt = jnp.concatenate([bbox, jnp.zeros((NGT, 4), jnp.float32)], axis=1)[:, :8]
    lbl = jnp.zeros((8, NGT), jnp.int32).at[0].set(label.astype(jnp.int32))

    ords = [jnp.asarray(o) for o in (_ORD1P, _ORD2P, _ORD1N, _ORD2N)]

    keepinfo, tab = pl.pallas_call(
        _tc_kernel,
        out_shape=[
            jax.ShapeDtypeStruct((8, NSAMP), jnp.int32),
            jax.ShapeDtypeStruct((16, NR, NL), jnp.float32),
        ],
        interpret=interpret,
    )(rois_pl, gt, lbl, *ords)

    tab16 = tab.reshape(16, NP).T.reshape(NP // 8, 128)
    cls, loc, sroi = _sc_assemble(keepinfo, tab16, interpret=interpret)
    return cls, loc, sroi


def kernel(bbox, label, rois):
    return _run(bbox, label, rois)


# speedup vs baseline: 1.1206x; 1.1206x over previous
import functools

import jax
import jax.numpy as jnp
import numpy as np
from jax.experimental import pallas as pl
from jax.experimental.pallas import tpu as pltpu
from jax.experimental.pallas import tpu_sc as plsc

N0 = 5064
NR, NL = 40, 128
NP = NR * NL
NGT = 64
NSAMP = 128
NWORK = 8


def _np_threefry(k0, k1, x0, x1):
    u32 = np.uint32
    rotations = ((13, 15, 26, 6), (17, 29, 16, 24))
    with np.errstate(over="ignore"):
        ks = (k0, k1, u32(k0 ^ k1 ^ u32(0x1BD11BDA)))
        x0 = (x0 + ks[0]).astype(u32)
        x1 = (x1 + ks[1]).astype(u32)
        for i in range(5):
            for r in rotations[i % 2]:
                x0 = (x0 + x1).astype(u32)
                x1 = ((x1 << u32(r)) | (x1 >> u32(32 - r))).astype(u32)
                x1 = (x0 ^ x1).astype(u32)
            x0 = (x0 + ks[(i + 1) % 3]).astype(u32)
            x1 = (x1 + ks[(i + 2) % 3] + u32(i + 1)).astype(u32)
    return x0, x1


def _np_split(kd):
    c1 = np.zeros(2, np.uint32)
    c2 = np.arange(2, dtype=np.uint32)
    b1, b2 = _np_threefry(kd[0], kd[1], c1, c2)
    return np.array([b1[0], b2[0]], np.uint32), np.array([b1[1], b2[1]], np.uint32)


def _np_bits(kd, n):
    j = np.arange(n, dtype=np.uint32)
    o0, o1 = _np_threefry(kd[0], kd[1], np.zeros(n, np.uint32), j)
    return o0 ^ o1


def _const_streams():
    out = []
    for seed in (1, 2):
        key = np.array([0, seed], np.uint32)
        key1, s1 = _np_split(key)
        _, s2 = _np_split(key1)
        for s in (s1, s2):
            b = _np_bits(s, N0)
            o = np.argsort(b, kind="stable").astype(np.int32)
            o = np.concatenate([o, np.full((NP - N0,), 100000, np.int32)])
            out.append(o.reshape(NR, NL))
    return out


_ORD1P, _ORD2P, _ORD1N, _ORD2N = _const_streams()


def _cumsum2d(x):
    for s in (1, 2, 4, 8, 16, 32, 64):
        x = x + jnp.concatenate(
            [jnp.zeros((NR, s), x.dtype), x[:, :NL - s]], axis=1)
    rt = x[:, NL - 1:NL]
    rc = rt
    for s in (1, 2, 4, 8, 16, 32):
        rc = rc + jnp.concatenate(
            [jnp.zeros((s, 1), x.dtype), rc[:NR - s, :]], axis=0)
    return x + (rc - rt)


def _bitsel(ordv, m, c, targets):
    cm = jnp.where(m, c, 0)
    hit = cm[None, :, :] == targets[:, None, None] + 1
    return jnp.sum(jnp.where(hit, ordv[None, :, :], 0), axis=(1, 2))


def _branch(mask, vm, idx2d, n, ord1, ord2, ktake):
    nm = (~mask) & vm
    cP = _cumsum2d(mask.astype(jnp.int32))
    cN = _cumsum2d(nm.astype(jnp.int32))
    p = jnp.where(mask, cP - 1, n + cN - 1)
    p = jnp.where(vm, p, 6000 + idx2d)
    rvec = jax.lax.iota(jnp.int32, ktake)
    m1 = ord1 < n
    c1 = _cumsum2d(m1.astype(jnp.int32))

    def one_round():
        return jnp.where(rvec < n, _bitsel(ord1, m1, c1, rvec), rvec)

    def two_rounds():
        m2 = ord2 < n
        c2 = _cumsum2d(m2.astype(jnp.int32))
        qt = jnp.where(rvec < n, _bitsel(ord2, m2, c2, rvec), rvec)
        return jnp.where(qt < n, _bitsel(ord1, m1, c1, qt), qt)

    pstar = jax.lax.cond(n > 1625, two_rounds, one_round)
    hit = p[None, :, :] == pstar[:, None, None]
    return jnp.sum(jnp.where(hit, idx2d[None, :, :], 0), axis=(1, 2))


def _tc_kernel(rois_ref, gt_ref, lbl_ref, o1p_ref, o2p_ref, o1n_ref, o2n_ref,
               keep_ref, tab_ref):
    x1 = rois_ref[0]
    y1 = rois_ref[1]
    x2 = rois_ref[2]
    y2 = rois_ref[3]
    gx1 = gt_ref[:, 0].reshape(NGT, 1, 1)
    gy1 = gt_ref[:, 1].reshape(NGT, 1, 1)
    gx2 = gt_ref[:, 2].reshape(NGT, 1, 1)
    gy2 = gt_ref[:, 3].reshape(NGT, 1, 1)

    iw = jnp.clip(jnp.minimum(gx2, x2[None]) - jnp.maximum(gx1, x1[None]), 0.0, None)
    ih = jnp.clip(jnp.minimum(gy2, y2[None]) - jnp.maximum(gy1, y1[None]), 0.0, None)
    inter = iw * ih
    area_r = (x2 - x1) * (y2 - y1)
    area_g = (gx2 - gx1) * (gy2 - gy1)
    union = area_r[None] + area_g - inter
    iou = inter / union

    maxv = jnp.max(iou, axis=0)
    g_iota = jax.lax.broadcasted_iota(jnp.int32, (NGT, NR, NL), 0)
    am = jnp.min(jnp.where(iou == maxv[None], g_iota, NGT), axis=0)

    idx2d = jax.lax.broadcasted_iota(jnp.int32, (NR, NL), 0) * NL + \
        jax.lax.broadcasted_iota(jnp.int32, (NR, NL), 1)
    vm = idx2d < N0
    pos_mask = (maxv >= 0.5) & vm
    neg_mask = (maxv < 0.5) & (maxv >= 0.0) & vm

    n_pos = jnp.sum(pos_mask.astype(jnp.int32))
    n_neg = jnp.sum(neg_mask.astype(jnp.int32))
    n_pos_t = jnp.minimum(jnp.sum(((maxv > 0.5) & vm).astype(jnp.int32)), 32)

    pos_roi = _branch(pos_mask, vm, idx2d, n_pos, o1p_ref[...], o2p_ref[...], 32)
    neg_roi = _branch(neg_mask, vm, idx2d, n_neg, o1n_ref[...], o2n_ref[...], 96)
    keep = jnp.concatenate([pos_roi, neg_roi])

    keep_ref[...] = jnp.concatenate(
        [keep.reshape(1, NSAMP),
         jnp.full((1, NSAMP), n_pos_t, jnp.int32),
         jnp.zeros((6, NSAMP), jnp.int32)], axis=0)

    hit2 = g_iota == am[None]

    def gsel(gv):
        return jnp.sum(jnp.where(hit2, gv, 0.0), axis=0)

    gx1d, gy1d, gx2d, gy2d = gsel(gx1), gsel(gy1), gsel(gx2), gsel(gy2)
    lblf = lbl_ref[0, :].astype(jnp.float32).reshape(NGT, 1, 1)
    clsf = gsel(lblf) + 1.0

    pw = x2 - x1
    ph = y2 - y1
    pcx = (x1 + x2) / 2.0
    pcy = (y1 + y2) / 2.0
    gw = gx2d - gx1d
    gh = gy2d - gy1d
    gcx = (gx1d + gx2d) / 2.0
    gcy = (gy1d + gy2d) / 2.0
    tx = (gcx - pcx) / pw
    ty = (gcy - pcy) / ph
    tw = jnp.log(gw / pw)
    th = jnp.log(gh / ph)

    tab_ref[...] = jnp.stack(
        [tx, ty, tw, th, x1, y1, x2, y2, clsf], axis=0)


def _sc_kernel(keep_h, tab_h, cls_h, loc_h, sroi_h,
               keep_v, npos_v, idx_v, rows_v, cls_v, loc_v, sroi_v, sem):
    w = jax.lax.axis_index("s") * 2 + jax.lax.axis_index("c")

    @pl.when(w < NWORK)
    def _():
        pltpu.sync_copy(keep_h.at[0, pl.ds(w * 16, 16)], keep_v)
        pltpu.sync_copy(keep_h.at[1, pl.ds(0, 16)], npos_v)
        keep = keep_v[...]
        r = jnp.right_shift(keep, 7)
        c = jnp.bitwise_and(keep, 127)
        lane = jax.lax.iota(jnp.int32, 16)
        for j in range(9):
            plsc.store_scatter(idx_v, [lane * 9 + j], r + j * NR)
        pltpu.async_copy(tab_h.at[idx_v], rows_v, sem).wait()

        def col(j):
            return plsc.load_gather(rows_v, [lane * 9 + j, c])

        tx, ty, tw, th = col(0), col(1), col(2), col(3)
        x1, y1, x2, y2 = col(4), col(5), col(6), col(7)
        clsi = col(8).astype(jnp.int32)

        sidx = w * 16 + lane
        cls = jnp.where(sidx < npos_v[...], clsi, 0)

        cls_v[...] = cls
        for j, v in enumerate((tx, ty, tw, th)):
            plsc.store_scatter(loc_v, [lane, jnp.full((16,), j, jnp.int32)], v)
        for j, v in enumerate((x1, y1, x2, y2)):
            plsc.store_scatter(sroi_v, [lane, jnp.full((16,), j, jnp.int32)], v)

        pltpu.sync_copy(cls_v, cls_h.at[pl.ds(w * 16, 16)])
        pltpu.sync_copy(loc_v, loc_h.at[pl.ds(w * 16, 16), :])
        pltpu.sync_copy(sroi_v, sroi_h.at[pl.ds(w * 16, 16), :])


def _sc_assemble(keepinfo, tab16, interpret=False):
    mesh = plsc.VectorSubcoreMesh(
        core_axis_name="c", subcore_axis_name="s", num_cores=2, num_subcores=16)
    f = pl.kernel(
        _sc_kernel,
        out_type=[
            jax.ShapeDtypeStruct((NSAMP,), jnp.int32),
            jax.ShapeDtypeStruct((NSAMP, 4), jnp.float32),
            jax.ShapeDtypeStruct((NSAMP, 4), jnp.float32),
        ],
        mesh=mesh,
        compiler_params=pltpu.CompilerParams(needs_layout_passes=False),
        scratch_types=[
            pltpu.VMEM((16,), jnp.int32),
            pltpu.VMEM((16,), jnp.int32),
            pltpu.VMEM((144,), jnp.int32),
            pltpu.VMEM((144, 128), jnp.float32),
            pltpu.VMEM((16,), jnp.int32),
            pltpu.VMEM((16, 4), jnp.float32),
            pltpu.VMEM((16, 4), jnp.float32),
            pltpu.SemaphoreType.DMA,
        ],
        interpret=interpret,
    )
    return f(keepinfo, tab16)


@functools.partial(jax.jit, static_argnames=("interpret",))
def _run(bbox, label, rois, interpret=False):
    bbox = bbox[0]
    label = label[0]
    rois_all = jnp.concatenate([rois, bbox], axis=0)
    rois_pad = jnp.concatenate(
        [rois_all, jnp.zeros((NP - N0, 4), jnp.float32)], axis=0)
    rois_pl = rois_pad.T.reshape(4, NR, NL)
    gt = jnp.concatenate([bbox, jnp.zeros((NGT, 4), jnp.float32)], axis=1)[:, :8]
    lbl = jnp.zeros((8, NGT), jnp.int32).at[0].set(label.astype(jnp.int32))

    ords = [jnp.asarray(o) for o in (_ORD1P, _ORD2P, _ORD1N, _ORD2N)]

    keepinfo, tab = pl.pallas_call(
        _tc_kernel,
        out_shape=[
            jax.ShapeDtypeStruct((8, NSAMP), jnp.int32),
            jax.ShapeDtypeStruct((9, NR, NL), jnp.float32),
        ],
        interpret=interpret,
    )(rois_pl, gt, lbl, *ords)

    cls, loc, sroi = _sc_assemble(
        keepinfo, tab.reshape(9 * NR, NL), interpret=interpret)
    return cls, loc, sroi


def kernel(bbox, label, rois):
    return _run(bbox, label, rois)
